# baseline (speedup 1.0000x reference)
import functools

import jax
import jax.numpy as jnp
from jax import lax
from jax.experimental import pallas as pl
from jax.experimental.pallas import tpu as pltpu
from jax.experimental.pallas import tpu_sc as plsc

S = 256
Y = 1000
B = 16384


YBLK = 8


def _k1_body(x_ref, o_ref):
    x = x_ref[...]
    m = jnp.max(x, axis=-1, keepdims=True)
    xs = x - m
    lse = jnp.log(jnp.sum(jnp.exp(xs), axis=-1, keepdims=True))
    y = xs - lse
    o_ref[...] = jnp.transpose(y, (1, 2, 0))


def _k1(T_logits):
    return pl.pallas_call(
        _k1_body,
        grid=(Y // YBLK,),
        in_specs=[pl.BlockSpec((S, YBLK, S), lambda i: (0, i, 0))],
        out_specs=pl.BlockSpec((YBLK, S, S), lambda i: (i, 0, 0)),
        out_shape=jax.ShapeDtypeStruct((Y, S, S), jnp.float32),
    )(T_logits)


NC, NS = 2, 16
NW = NC * NS
BPW = B // NW
CHUNK = 128
NCHUNK = BPW // CHUNK


def _k2(table, idx3):
    mesh = plsc.VectorSubcoreMesh(core_axis_name="c", subcore_axis_name="s")

    @functools.partial(
        pl.kernel,
        mesh=mesh,
        out_type=jax.ShapeDtypeStruct((B, S), jnp.float32),
        scratch_types=[
            pltpu.VMEM((NCHUNK, CHUNK), jnp.int32),
            pltpu.VMEM((CHUNK, S), jnp.float32),
            pltpu.SemaphoreType.DMA,
        ],
    )
    def gather_kernel(table_hbm, idx_hbm, out_hbm, idx_v, rows_v, sem):
        wid = lax.axis_index("s") * NC + lax.axis_index("c")
        base = wid * BPW
        pltpu.sync_copy(idx_hbm.at[wid], idx_v)
        for c in range(NCHUNK):
            pltpu.async_copy(table_hbm.at[idx_v.at[c]], rows_v, sem).wait()
            pltpu.sync_copy(rows_v, out_hbm.at[pl.ds(base + c * CHUNK, CHUNK)])

    return gather_kernel(table, idx3)


TBLK = 512


def _k3_body(x_ref, o_ref):
    o_ref[...] = jnp.transpose(x_ref[...])


def _k3(out_T):
    return pl.pallas_call(
        _k3_body,
        grid=(B // TBLK,),
        in_specs=[pl.BlockSpec((TBLK, S), lambda i: (i, 0))],
        out_specs=pl.BlockSpec((S, TBLK), lambda i: (0, i)),
        out_shape=jax.ShapeDtypeStruct((S, B), jnp.float32),
    )(out_T)


@jax.jit
def kernel(T_logits, symbol_idx, state_idx):
    T_t = _k1(T_logits).reshape(Y * S, S)
    idx = symbol_idx.astype(jnp.int32) * S + state_idx.astype(jnp.int32)
    idx3 = idx.reshape(NW, NCHUNK, CHUNK)
    out_T = _k2(T_t, idx3)
    return _k3(out_T)


# speedup vs baseline: 4.9579x; 4.9579x over previous
import functools

import jax
import jax.numpy as jnp
from jax import lax
from jax.experimental import pallas as pl
from jax.experimental.pallas import tpu as pltpu
from jax.experimental.pallas import tpu_sc as plsc

S = 256
Y = 1000
B = 16384


YBLK = 8


def _k1_body(x_ref, o_ref):
    for i in range(YBLK):
        x = x_ref[:, i * S:(i + 1) * S]
        m = jnp.max(x, axis=-1, keepdims=True)
        xs = x - m
        lse = jnp.log(jnp.sum(jnp.exp(xs), axis=-1, keepdims=True))
        o_ref[i] = jnp.transpose(xs - lse)


def _k1(T2):
    return pl.pallas_call(
        _k1_body,
        grid=(Y // YBLK,),
        in_specs=[pl.BlockSpec((S, YBLK * S), lambda i: (0, i))],
        out_specs=pl.BlockSpec((YBLK, S, S), lambda i: (i, 0, 0)),
        out_shape=jax.ShapeDtypeStruct((Y, S, S), jnp.float32),
    )(T2)


NC, NS = 2, 16
NW = NC * NS
BPW = B // NW
CHUNK = 128
NCHUNK = BPW // CHUNK


def _k2(table, idx3):
    mesh = plsc.VectorSubcoreMesh(core_axis_name="c", subcore_axis_name="s")

    @functools.partial(
        pl.kernel,
        mesh=mesh,
        out_type=jax.ShapeDtypeStruct((B, S), jnp.float32),
        scratch_types=[
            pltpu.VMEM((NCHUNK, CHUNK), jnp.int32),
            pltpu.VMEM((CHUNK, S), jnp.float32),
            pltpu.SemaphoreType.DMA,
        ],
    )
    def gather_kernel(table_hbm, idx_hbm, out_hbm, idx_v, rows_v, sem):
        wid = lax.axis_index("s") * NC + lax.axis_index("c")
        base = wid * BPW
        pltpu.sync_copy(idx_hbm.at[wid], idx_v)
        for c in range(NCHUNK):
            pltpu.async_copy(table_hbm.at[idx_v.at[c]], rows_v, sem).wait()
            pltpu.sync_copy(rows_v, out_hbm.at[pl.ds(base + c * CHUNK, CHUNK)])

    return gather_kernel(table, idx3)


TBLK = 512


def _k3_body(x_ref, o_ref):
    o_ref[...] = jnp.transpose(x_ref[...])


def _k3(out_T):
    return pl.pallas_call(
        _k3_body,
        grid=(B // TBLK,),
        in_specs=[pl.BlockSpec((TBLK, S), lambda i: (i, 0))],
        out_specs=pl.BlockSpec((S, TBLK), lambda i: (0, i)),
        out_shape=jax.ShapeDtypeStruct((S, B), jnp.float32),
    )(out_T)


@jax.jit
def kernel(T_logits, symbol_idx, state_idx):
    T_t = _k1(T_logits.reshape(S, Y * S)).reshape(Y * S, S)
    idx = symbol_idx.astype(jnp.int32) * S + state_idx.astype(jnp.int32)
    idx3 = idx.reshape(NW, NCHUNK, CHUNK)
    out_T = _k2(T_t, idx3)
    return _k3(out_T)
